# baseline (speedup 1.0000x reference)
import functools

import jax
import jax.numpy as jnp
from jax import lax
from jax.experimental import pallas as pl
from jax.experimental.pallas import tpu as pltpu
from jax.experimental.pallas import tpu_sc as plsc

_CAP = 65536
_DIM = 256
_BATCH = 4096
_NC = 2
_NS = 16
_NW = _NC * _NS
_ROWS = _CAP // _NW
_CHUNK = 168
_NBUF = 3

_CHUNKS = []
_off = 0
while _off < _ROWS:
    _CHUNKS.append((_off, min(_CHUNK, _ROWS - _off)))
    _off += _CHUNK
_NCHUNK = len(_CHUNKS)


def _fifo_body(x_hbm, mem_hbm, out_hbm, buf, *sems):
    in_sems = sems[:_NBUF]
    out_sems = sems[_NBUF:]
    c = lax.axis_index("c")
    s = lax.axis_index("s")
    wid = s * _NC + c
    base = wid * _ROWS

    def run(src_hbm, src_base):
        def start_in(g):
            off, rows = _CHUNKS[g]
            b = g % _NBUF
            return pltpu.async_copy(
                src_hbm.at[pl.ds(src_base + off, rows)],
                buf.at[pl.ds(b * _CHUNK, rows)],
                in_sems[b],
            )

        def start_out(g):
            off, rows = _CHUNKS[g]
            b = g % _NBUF
            return pltpu.async_copy(
                buf.at[pl.ds(b * _CHUNK, rows)],
                out_hbm.at[pl.ds(base + off, rows)],
                out_sems[b],
            )

        ins = [start_in(g) for g in range(min(_NBUF, _NCHUNK))]
        outs = [None] * _NCHUNK
        for g in range(_NCHUNK):
            ins[g].wait()
            outs[g] = start_out(g)
            nxt = g + _NBUF
            if nxt < _NCHUNK:
                outs[g].wait()
                ins.append(start_in(nxt))
        for g in range(max(0, _NCHUNK - _NBUF), _NCHUNK):
            outs[g].wait()

    @pl.when(base < _BATCH)
    def _copy_x():
        run(x_hbm, base)

    @pl.when(base >= _BATCH)
    def _copy_mem():
        run(mem_hbm, base - _BATCH)


def kernel(x, classes, memory):
    del classes
    run = functools.partial(
        pl.kernel,
        mesh=plsc.VectorSubcoreMesh(core_axis_name="c", subcore_axis_name="s"),
        out_type=jax.ShapeDtypeStruct((_CAP, _DIM), jnp.float32),
        scratch_types=(
            [pltpu.VMEM((_NBUF * _CHUNK, _DIM), jnp.float32)]
            + [pltpu.SemaphoreType.DMA] * (2 * _NBUF)
        ),
    )(_fifo_body)
    return run(x, memory)


# speedup vs baseline: 1.0018x; 1.0018x over previous
import functools

import jax
import jax.numpy as jnp
from jax import lax
from jax.experimental import pallas as pl
from jax.experimental.pallas import tpu as pltpu
from jax.experimental.pallas import tpu_sc as plsc

_CAP = 65536
_DIM = 256
_BATCH = 4096
_NC = 2
_NS = 16
_NW = _NC * _NS
_ROWS = _CAP // _NW
_CHUNK = 120
_NBUF = 4
_AHEAD = 2

_CHUNKS = []
_off = 0
while _off < _ROWS:
    _CHUNKS.append((_off, min(_CHUNK, _ROWS - _off)))
    _off += _CHUNK
_NCHUNK = len(_CHUNKS)


def _fifo_body(x_hbm, mem_hbm, out_hbm, buf, *sems):
    in_sems = sems[:_NBUF]
    out_sems = sems[_NBUF:]
    c = lax.axis_index("c")
    s = lax.axis_index("s")
    wid = s * _NC + c
    base = wid * _ROWS

    def run(src_hbm, src_base):
        def start_in(g):
            off, rows = _CHUNKS[g]
            b = g % _NBUF
            return pltpu.async_copy(
                src_hbm.at[pl.ds(src_base + off, rows)],
                buf.at[pl.ds(b * _CHUNK, rows)],
                in_sems[b],
            )

        def start_out(g):
            off, rows = _CHUNKS[g]
            b = g % _NBUF
            return pltpu.async_copy(
                buf.at[pl.ds(b * _CHUNK, rows)],
                out_hbm.at[pl.ds(base + off, rows)],
                out_sems[b],
            )

        ins = [start_in(g) for g in range(min(_AHEAD, _NCHUNK))]
        outs = [None] * _NCHUNK
        waited = [False] * _NCHUNK
        for g in range(_NCHUNK):
            ins[g].wait()
            outs[g] = start_out(g)
            nxt = g + _AHEAD
            if nxt < _NCHUNK:
                prev = nxt - _NBUF
                if prev >= 0:
                    outs[prev].wait()
                    waited[prev] = True
                ins.append(start_in(nxt))
        for g in range(_NCHUNK):
            if not waited[g]:
                outs[g].wait()

    @pl.when(base < _BATCH)
    def _copy_x():
        run(x_hbm, base)

    @pl.when(base >= _BATCH)
    def _copy_mem():
        run(mem_hbm, base - _BATCH)


def kernel(x, classes, memory):
    del classes
    run = functools.partial(
        pl.kernel,
        mesh=plsc.VectorSubcoreMesh(core_axis_name="c", subcore_axis_name="s"),
        out_type=jax.ShapeDtypeStruct((_CAP, _DIM), jnp.float32),
        scratch_types=(
            [pltpu.VMEM((_NBUF * _CHUNK, _DIM), jnp.float32)]
            + [pltpu.SemaphoreType.DMA] * (2 * _NBUF)
        ),
    )(_fifo_body)
    return run(x, memory)


# speedup vs baseline: 1.0087x; 1.0069x over previous
import functools

import jax
import jax.numpy as jnp
from jax import lax
from jax.experimental import pallas as pl
from jax.experimental.pallas import tpu as pltpu
from jax.experimental.pallas import tpu_sc as plsc

_CAP = 65536
_DIM = 256
_BATCH = 4096
_NC = 2
_NS = 16
_NW = _NC * _NS
_ROWS = _CAP // _NW
_CHUNK = 248
_NBUF = 2
_AHEAD = 2

_CHUNKS = []
_off = 0
while _off < _ROWS:
    _CHUNKS.append((_off, min(_CHUNK, _ROWS - _off)))
    _off += _CHUNK
_NCHUNK = len(_CHUNKS)


def _fifo_body(x_hbm, mem_hbm, out_hbm, buf, *sems):
    in_sems = sems[:_NBUF]
    out_sems = sems[_NBUF:]
    c = lax.axis_index("c")
    s = lax.axis_index("s")
    wid = s * _NC + c
    base = wid * _ROWS

    def run(src_hbm, src_base):
        def start_in(g):
            off, rows = _CHUNKS[g]
            b = g % _NBUF
            return pltpu.async_copy(
                src_hbm.at[pl.ds(src_base + off, rows)],
                buf.at[pl.ds((s * _NBUF + b) * _CHUNK, rows)],
                in_sems[b],
            )

        def start_out(g):
            off, rows = _CHUNKS[g]
            b = g % _NBUF
            return pltpu.async_copy(
                buf.at[pl.ds((s * _NBUF + b) * _CHUNK, rows)],
                out_hbm.at[pl.ds(base + off, rows)],
                out_sems[b],
            )

        ins = [start_in(g) for g in range(min(_AHEAD, _NCHUNK))]
        outs = [None] * _NCHUNK
        waited = [False] * _NCHUNK
        for g in range(_NCHUNK):
            ins[g].wait()
            outs[g] = start_out(g)
            nxt = g + _AHEAD
            if nxt < _NCHUNK:
                prev = nxt - _NBUF
                if prev >= 0:
                    outs[prev].wait()
                    waited[prev] = True
                ins.append(start_in(nxt))
        for g in range(_NCHUNK):
            if not waited[g]:
                outs[g].wait()

    @pl.when(base < _BATCH)
    def _copy_x():
        run(x_hbm, base)

    @pl.when(base >= _BATCH)
    def _copy_mem():
        run(mem_hbm, base - _BATCH)


def kernel(x, classes, memory):
    del classes
    run = functools.partial(
        pl.kernel,
        mesh=plsc.VectorSubcoreMesh(core_axis_name="c", subcore_axis_name="s"),
        out_type=jax.ShapeDtypeStruct((_CAP, _DIM), jnp.float32),
        scratch_types=(
            [pltpu.VMEM_SHARED((_NS * _NBUF * _CHUNK, _DIM), jnp.float32)]
            + [pltpu.SemaphoreType.DMA] * (2 * _NBUF)
        ),
    )(_fifo_body)
    return run(x, memory)
